# baseline (speedup 1.0000x reference)
import functools

import jax
import jax.numpy as jnp
from jax import lax
from jax.experimental import pallas as pl
from jax.experimental.pallas import tpu as pltpu
from jax.experimental.pallas import tpu_sc as plsc

DIM = 64
LANES = 16
CHUNK = 128


def kernel(u, i, j, user_factors, item_factors, item_biases):
    B = u.shape[0]
    info = plsc.get_sparse_core_info()
    nw = info.num_cores * info.num_subcores
    bpw = B // nw
    n_chunks = bpw // CHUNK

    n_items = item_factors.shape[0]
    half_items = n_items // 2
    if_a = item_factors[:half_items].reshape(-1, 2 * DIM)
    if_b = item_factors[half_items:].reshape(-1, 2 * DIM)
    uf2 = user_factors.reshape(-1, 2 * DIM)
    ib1 = item_biases.reshape(-1)
    half_pairs = half_items // 2

    mesh = plsc.VectorSubcoreMesh(core_axis_name="c", subcore_axis_name="s")

    @functools.partial(
        pl.kernel,
        mesh=mesh,
        out_type=jax.ShapeDtypeStruct((B,), jnp.float32),
        scratch_types=[
            pltpu.VMEM((bpw,), jnp.int32),
            pltpu.VMEM((bpw,), jnp.int32),
            pltpu.VMEM((bpw,), jnp.int32),
            pltpu.VMEM((bpw,), jnp.int32),
            pltpu.VMEM((bpw,), jnp.int32),
            pltpu.VMEM((bpw,), jnp.int32),
            pltpu.VMEM((bpw,), jnp.int32),
            pltpu.VMEM((bpw,), jnp.int32),
            pltpu.VMEM((CHUNK, 2 * DIM), jnp.float32),
            pltpu.VMEM((CHUNK, 2 * DIM), jnp.float32),
            pltpu.VMEM((CHUNK, 2 * DIM), jnp.float32),
            pltpu.VMEM((CHUNK, 2 * DIM), jnp.float32),
            pltpu.VMEM((CHUNK, 2 * DIM), jnp.float32),
            pltpu.VMEM((bpw,), jnp.float32),
            pltpu.VMEM((bpw,), jnp.float32),
            pltpu.VMEM((bpw,), jnp.float32),
            pltpu.SemaphoreType.DMA,
        ],
    )
    def sc_kernel(u_hbm, i_hbm, j_hbm, ufa_hbm, ifa_hbm, ifb_hbm, ib_hbm,
                  out_hbm,
                  u_idx, i_idx, j_idx, u_sh, i_sa, i_sb, j_sa, j_sb,
                  u_rows, ia_rows, ib_rows, ja_rows, jb_rows,
                  bi_v, bj_v, out_v, sem):
        wid = lax.axis_index("s") * info.num_cores + lax.axis_index("c")
        base = wid * bpw

        pltpu.sync_copy(u_hbm.at[pl.ds(base, bpw)], u_idx)
        pltpu.sync_copy(i_hbm.at[pl.ds(base, bpw)], i_idx)
        pltpu.sync_copy(j_hbm.at[pl.ds(base, bpw)], j_idx)

        maxa = jnp.full((LANES,), half_pairs - 1, jnp.int32)
        halfp = jnp.full((LANES,), half_pairs, jnp.int32)
        zero16 = jnp.zeros((LANES,), jnp.int32)

        def shift_body(g, carry):
            sl = pl.ds(g * LANES, LANES)
            u_sh[sl] = lax.shift_right_logical(u_idx[sl], 1)
            ip = lax.shift_right_logical(i_idx[sl], 1)
            jp = lax.shift_right_logical(j_idx[sl], 1)
            i_sa[sl] = jnp.minimum(ip, maxa)
            i_sb[sl] = jnp.maximum(ip - halfp, zero16)
            j_sa[sl] = jnp.minimum(jp, maxa)
            j_sb[sl] = jnp.maximum(jp - halfp, zero16)
            return carry

        lax.fori_loop(0, bpw // LANES, shift_body, 0)

        bias_copies = []
        for c in range(n_chunks):
            sl = pl.ds(c * CHUNK, CHUNK)
            bias_copies.append(pltpu.async_copy(
                ib_hbm.at[i_idx.at[sl]], bi_v.at[sl], sem))
            bias_copies.append(pltpu.async_copy(
                ib_hbm.at[j_idx.at[sl]], bj_v.at[sl], sem))
        for cp in bias_copies:
            cp.wait()

        lane_iota = lax.iota(jnp.int32, LANES)
        perms = [jnp.bitwise_xor(lane_iota, jnp.full((LANES,), s, jnp.int32))
                 for s in (1, 2, 4, 8)]
        one16 = jnp.full((LANES,), 1, jnp.int32)
        halfi = jnp.full((LANES,), half_items, jnp.int32)

        def chunk_body(c, carry):
            sl = pl.ds(c * CHUNK, CHUNK)
            copies = [
                pltpu.async_copy(ufa_hbm.at[u_sh.at[sl]], u_rows, sem),
                pltpu.async_copy(ifa_hbm.at[i_sa.at[sl]], ia_rows, sem),
                pltpu.async_copy(ifb_hbm.at[i_sb.at[sl]], ib_rows, sem),
                pltpu.async_copy(ifa_hbm.at[j_sa.at[sl]], ja_rows, sem),
                pltpu.async_copy(ifb_hbm.at[j_sb.at[sl]], jb_rows, sem),
            ]
            for cp in copies:
                cp.wait()

            def group_body(gg, carry2):
                gb = c * CHUNK + gg * LANES
                gsl = pl.ds(gb, LANES)
                pu = jnp.bitwise_and(u_idx[gsl], one16).astype(jnp.float32)
                ivec = i_idx[gsl]
                jvec = j_idx[gsl]
                pi = jnp.bitwise_and(ivec, one16).astype(jnp.float32)
                pj = jnp.bitwise_and(jvec, one16).astype(jnp.float32)
                hi_half = jnp.minimum(
                    jnp.maximum(ivec - (halfi - one16), zero16),
                    one16).astype(jnp.float32)
                hj_half = jnp.minimum(
                    jnp.maximum(jvec - (halfi - one16), zero16),
                    one16).astype(jnp.float32)
                acc = bi_v[gsl] - bj_v[gsl]
                for ee in range(LANES):
                    e = gg * LANES + ee
                    lane = jnp.full((LANES,), ee, jnp.int32)
                    fu = jnp.take(pu, lane)
                    fi = jnp.take(pi, lane)
                    fj = jnp.take(pj, lane)
                    hi = jnp.take(hi_half, lane)
                    hj = jnp.take(hj_half, lane)
                    p = None
                    for k in range(DIM // LANES):
                        lo = pl.ds(k * LANES, LANES)
                        hisl = pl.ds(DIM + k * LANES, LANES)
                        ul = u_rows[e, lo]
                        uv = ul + fu * (u_rows[e, hisl] - ul)
                        ial = ia_rows[e, lo]
                        ibl = ib_rows[e, lo]
                        il = ial + hi * (ibl - ial)
                        iah = ia_rows[e, hisl]
                        ibh = ib_rows[e, hisl]
                        ih = iah + hi * (ibh - iah)
                        iv = il + fi * (ih - il)
                        jal = ja_rows[e, lo]
                        jbl = jb_rows[e, lo]
                        jl = jal + hj * (jbl - jal)
                        jah = ja_rows[e, hisl]
                        jbh = jb_rows[e, hisl]
                        jh = jah + hj * (jbh - jah)
                        jv = jl + fj * (jh - jl)
                        t = uv * (iv - jv)
                        p = t if p is None else p + t
                    for perm in perms:
                        p = p + jnp.take(p, perm)
                    acc = jnp.where(lane_iota == ee, p + acc, acc)
                out_v[pl.ds(gb, LANES)] = acc
                return carry2

            lax.fori_loop(0, CHUNK // LANES, group_body, 0)
            return carry

        lax.fori_loop(0, n_chunks, chunk_body, 0)

        pltpu.sync_copy(out_v, out_hbm.at[pl.ds(base, bpw)])

    return sc_kernel(u, i, j, uf2, if_a, if_b, ib1)


# speedup vs baseline: 1.5879x; 1.5879x over previous
import functools

import jax
import jax.numpy as jnp
from jax import lax
from jax.experimental import pallas as pl
from jax.experimental.pallas import tpu as pltpu
from jax.experimental.pallas import tpu_sc as plsc

DIM = 64
LANES = 16
CHUNK = 128


def kernel(u, i, j, user_factors, item_factors, item_biases):
    B = u.shape[0]
    info = plsc.get_sparse_core_info()
    nw = info.num_cores * info.num_subcores
    bpw = B // nw
    n_chunks = bpw // CHUNK

    n_items = item_factors.shape[0]
    half_items = n_items // 2
    if_a = item_factors[:half_items].reshape(-1, 2 * DIM)
    if_b = item_factors[half_items:].reshape(-1, 2 * DIM)
    uf2 = user_factors.reshape(-1, 2 * DIM)
    ib1 = item_biases.reshape(-1)
    half_pairs = half_items // 2

    mesh = plsc.VectorSubcoreMesh(core_axis_name="c", subcore_axis_name="s")

    @functools.partial(
        pl.kernel,
        mesh=mesh,
        out_type=jax.ShapeDtypeStruct((B,), jnp.float32),
        scratch_types=[
            pltpu.VMEM((bpw,), jnp.int32),
            pltpu.VMEM((bpw,), jnp.int32),
            pltpu.VMEM((bpw,), jnp.int32),
            pltpu.VMEM((bpw,), jnp.int32),
            pltpu.VMEM((bpw,), jnp.int32),
            pltpu.VMEM((bpw,), jnp.int32),
            pltpu.VMEM((bpw,), jnp.int32),
            pltpu.VMEM((bpw,), jnp.int32),
            pltpu.VMEM((CHUNK, 2 * DIM), jnp.float32),
            pltpu.VMEM((CHUNK, 2 * DIM), jnp.float32),
            pltpu.VMEM((CHUNK, 2 * DIM), jnp.float32),
            pltpu.VMEM((CHUNK, 2 * DIM), jnp.float32),
            pltpu.VMEM((CHUNK, 2 * DIM), jnp.float32),
            pltpu.VMEM((bpw,), jnp.float32),
            pltpu.VMEM((bpw,), jnp.float32),
            pltpu.VMEM((bpw,), jnp.float32),
            pltpu.SemaphoreType.DMA,
        ],
    )
    def sc_kernel(u_hbm, i_hbm, j_hbm, ufa_hbm, ifa_hbm, ifb_hbm, ib_hbm,
                  out_hbm,
                  u_idx, i_idx, j_idx, u_sh, i_sa, i_sb, j_sa, j_sb,
                  u_rows, ia_rows, ib_rows, ja_rows, jb_rows,
                  bi_v, bj_v, out_v, sem):
        wid = lax.axis_index("s") * info.num_cores + lax.axis_index("c")
        base = wid * bpw

        pltpu.sync_copy(u_hbm.at[pl.ds(base, bpw)], u_idx)
        pltpu.sync_copy(i_hbm.at[pl.ds(base, bpw)], i_idx)
        pltpu.sync_copy(j_hbm.at[pl.ds(base, bpw)], j_idx)

        maxa = jnp.full((LANES,), half_pairs - 1, jnp.int32)
        halfp = jnp.full((LANES,), half_pairs, jnp.int32)
        zero16 = jnp.zeros((LANES,), jnp.int32)

        def shift_body(g, carry):
            sl = pl.ds(g * LANES, LANES)
            u_sh[sl] = lax.shift_right_logical(u_idx[sl], 1)
            ip = lax.shift_right_logical(i_idx[sl], 1)
            jp = lax.shift_right_logical(j_idx[sl], 1)
            i_sa[sl] = maxa - jnp.minimum(jnp.abs(ip - maxa), maxa)
            i_sb[sl] = jnp.minimum(jnp.abs(ip - halfp), maxa)
            j_sa[sl] = maxa - jnp.minimum(jnp.abs(jp - maxa), maxa)
            j_sb[sl] = jnp.minimum(jnp.abs(jp - halfp), maxa)
            return carry

        lax.fori_loop(0, bpw // LANES, shift_body, 0)

        bias_copies = []
        for c in range(n_chunks):
            sl = pl.ds(c * CHUNK, CHUNK)
            bias_copies.append(pltpu.async_copy(
                ib_hbm.at[i_idx.at[sl]], bi_v.at[sl], sem))
            bias_copies.append(pltpu.async_copy(
                ib_hbm.at[j_idx.at[sl]], bj_v.at[sl], sem))
        for cp in bias_copies:
            cp.wait()

        lane_iota = lax.iota(jnp.int32, LANES)
        perms = [jnp.bitwise_xor(lane_iota, jnp.full((LANES,), s, jnp.int32))
                 for s in (1, 2, 4, 8)]
        one16 = jnp.full((LANES,), 1, jnp.int32)
        halfi = jnp.full((LANES,), half_items, jnp.int32)

        def chunk_body(c, carry):
            sl = pl.ds(c * CHUNK, CHUNK)
            copies = [
                pltpu.async_copy(ufa_hbm.at[u_sh.at[sl]], u_rows, sem),
                pltpu.async_copy(ifa_hbm.at[i_sa.at[sl]], ia_rows, sem),
                pltpu.async_copy(ifb_hbm.at[i_sb.at[sl]], ib_rows, sem),
                pltpu.async_copy(ifa_hbm.at[j_sa.at[sl]], ja_rows, sem),
                pltpu.async_copy(ifb_hbm.at[j_sb.at[sl]], jb_rows, sem),
            ]
            for cp in copies:
                cp.wait()

            def group_body(gg, carry2):
                gb = c * CHUNK + gg * LANES
                gsl = pl.ds(gb, LANES)
                pu = jnp.bitwise_and(u_idx[gsl], one16).astype(jnp.float32)
                ivec = i_idx[gsl]
                jvec = j_idx[gsl]
                pi = jnp.bitwise_and(ivec, one16).astype(jnp.float32)
                pj = jnp.bitwise_and(jvec, one16).astype(jnp.float32)
                hi_half = jnp.minimum(
                    jnp.maximum(ivec - (halfi - one16), zero16),
                    one16).astype(jnp.float32)
                hj_half = jnp.minimum(
                    jnp.maximum(jvec - (halfi - one16), zero16),
                    one16).astype(jnp.float32)
                acc = bi_v[gsl] - bj_v[gsl]
                for ee in range(LANES):
                    e = gg * LANES + ee
                    lane = jnp.full((LANES,), ee, jnp.int32)
                    fu = jnp.take(pu, lane)
                    fi = jnp.take(pi, lane)
                    fj = jnp.take(pj, lane)
                    hi = jnp.take(hi_half, lane)
                    hj = jnp.take(hj_half, lane)
                    p = None
                    for k in range(DIM // LANES):
                        lo = pl.ds(k * LANES, LANES)
                        hisl = pl.ds(DIM + k * LANES, LANES)
                        ul = u_rows[e, lo]
                        uv = ul + fu * (u_rows[e, hisl] - ul)
                        ial = ia_rows[e, lo]
                        ibl = ib_rows[e, lo]
                        il = ial + hi * (ibl - ial)
                        iah = ia_rows[e, hisl]
                        ibh = ib_rows[e, hisl]
                        ih = iah + hi * (ibh - iah)
                        iv = il + fi * (ih - il)
                        jal = ja_rows[e, lo]
                        jbl = jb_rows[e, lo]
                        jl = jal + hj * (jbl - jal)
                        jah = ja_rows[e, hisl]
                        jbh = jb_rows[e, hisl]
                        jh = jah + hj * (jbh - jah)
                        jv = jl + fj * (jh - jl)
                        t = uv * (iv - jv)
                        p = t if p is None else p + t
                    for perm in perms:
                        p = p + jnp.take(p, perm)
                    acc = jnp.where(lane_iota == ee, p + acc, acc)
                out_v[pl.ds(gb, LANES)] = acc
                return carry2

            lax.fori_loop(0, CHUNK // LANES, group_body, 0)
            return carry

        lax.fori_loop(0, n_chunks, chunk_body, 0)

        pltpu.sync_copy(out_v, out_hbm.at[pl.ds(base, bpw)])

    return sc_kernel(u, i, j, uf2, if_a, if_b, ib1)


# speedup vs baseline: 3.6339x; 2.2885x over previous
import functools

import jax
import jax.numpy as jnp
from jax import lax
from jax.experimental import pallas as pl
from jax.experimental.pallas import tpu as pltpu
from jax.experimental.pallas import tpu_sc as plsc

DIM = 64
LANES = 16
NSEM = 8


def kernel(u, i, j, user_factors, item_factors, item_biases):
    B = u.shape[0]
    info = plsc.get_sparse_core_info()
    nw = info.num_cores * info.num_subcores
    bpw = B // nw

    ib1 = item_biases.reshape(-1)

    mesh = plsc.VectorSubcoreMesh(core_axis_name="c", subcore_axis_name="s")

    @functools.partial(
        pl.kernel,
        mesh=mesh,
        out_type=jax.ShapeDtypeStruct((B,), jnp.float32),
        scratch_types=[
            pltpu.VMEM((bpw,), jnp.int32),
            pltpu.VMEM((bpw,), jnp.int32),
            pltpu.VMEM((bpw,), jnp.int32),
            pltpu.VMEM((bpw // 2, DIM), jnp.float32),
            pltpu.VMEM((bpw // 2, DIM), jnp.float32),
            pltpu.VMEM((bpw // 2, DIM), jnp.float32),
            pltpu.VMEM((bpw,), jnp.float32),
            pltpu.VMEM((bpw,), jnp.float32),
            pltpu.VMEM((bpw,), jnp.float32),
            [pltpu.SemaphoreType.DMA] * NSEM,
        ],
    )
    def sc_kernel(u_hbm, i_hbm, j_hbm, uf_hbm, if_hbm, ib_hbm, out_hbm,
                  u_idx, i_idx, j_idx,
                  u_rows, i_rows, j_rows, bi_v, bj_v, out_v, sems):
        wid = lax.axis_index("s") * info.num_cores + lax.axis_index("c")
        base = wid * bpw

        pltpu.sync_copy(u_hbm.at[pl.ds(base, bpw)], u_idx)
        pltpu.sync_copy(i_hbm.at[pl.ds(base, bpw)], i_idx)
        pltpu.sync_copy(j_hbm.at[pl.ds(base, bpw)], j_idx)

        bias_copies = []
        for c in range(bpw // 128):
            sl = pl.ds(c * 128, 128)
            bias_copies.append(pltpu.async_copy(
                ib_hbm.at[i_idx.at[sl]], bi_v.at[sl], sems[0]))
            bias_copies.append(pltpu.async_copy(
                ib_hbm.at[j_idx.at[sl]], bj_v.at[sl], sems[1]))
        for cp in bias_copies:
            cp.wait()

        lane_iota = lax.iota(jnp.int32, LANES)
        perms = [jnp.bitwise_xor(lane_iota, jnp.full((LANES,), s, jnp.int32))
                 for s in (1, 2, 4, 8)]

        half = bpw // 2
        rows_per_sem = half * 3 // NSEM
        for h in range(2):
            hb = h * half

            def fetch_body(g, carry):
                gb = g * LANES
                uvec = u_idx[pl.ds(hb + gb, LANES)]
                ivec = i_idx[pl.ds(hb + gb, LANES)]
                jvec = j_idx[pl.ds(hb + gb, LANES)]
                for ee in range(LANES):
                    e = gb + ee
                    s0 = (ee * 3) % NSEM
                    pltpu.async_copy(uf_hbm.at[uvec[ee]], u_rows.at[e],
                                     sems[s0])
                    pltpu.async_copy(if_hbm.at[ivec[ee]], i_rows.at[e],
                                     sems[(s0 + 1) % NSEM])
                    pltpu.async_copy(if_hbm.at[jvec[ee]], j_rows.at[e],
                                     sems[(s0 + 2) % NSEM])
                return carry

            lax.fori_loop(0, half // LANES, fetch_body, 0)

            for s in range(NSEM):
                pltpu.make_async_copy(
                    uf_hbm.at[pl.ds(0, rows_per_sem)],
                    u_rows.at[pl.ds(0, rows_per_sem)],
                    sems[s]).wait()

            def group_body(gg, carry):
                gb = gg * LANES
                acc = (bi_v[pl.ds(hb + gb, LANES)]
                       - bj_v[pl.ds(hb + gb, LANES)])
                for ee in range(LANES):
                    e = gb + ee
                    p = None
                    for k in range(DIM // LANES):
                        ksl = pl.ds(k * LANES, LANES)
                        t = u_rows[e, ksl] * (i_rows[e, ksl] - j_rows[e, ksl])
                        p = t if p is None else p + t
                    for perm in perms:
                        p = p + jnp.take(p, perm)
                    acc = jnp.where(lane_iota == ee, p + acc, acc)
                out_v[pl.ds(hb + gb, LANES)] = acc
                return carry

            lax.fori_loop(0, half // LANES, group_body, 0)

        pltpu.sync_copy(out_v, out_hbm.at[pl.ds(base, bpw)])

    return sc_kernel(u, i, j, user_factors, item_factors, ib1)
